# Initial kernel scaffold; baseline (speedup 1.0000x reference)
import jax
import jax.numpy as jnp
from jax.experimental import pallas as pl


def kernel(x, W_enc, W_dec):
    raise NotImplementedError("write your pallas kernel here")


# speedup vs baseline: 9.4057x; 9.4057x over previous
import jax
import jax.numpy as jnp
from jax.experimental import pallas as pl

HIDDEN = 2048
LATENT = 16384
TOPK = 64
NTOK = 2048

BM = 256
LC = 1024
BR = 128
KC = 512
BISECT_ITERS = 32


def _enc_body(x_ref, w_ref, out_ref):
    out_ref[...] = jax.lax.dot_general(
        x_ref[...], w_ref[...],
        (((1,), (1,)), ((), ())),
        preferred_element_type=jnp.float32,
        precision=jax.lax.Precision.DEFAULT)


def _topk_body(z_ref, out_ref):
    z = z_ref[...]
    lo = jnp.min(z, axis=1, keepdims=True)
    hi = jnp.max(z, axis=1, keepdims=True) + 0.5

    def body(i, carry):
        lo, hi = carry
        mid = 0.5 * (lo + hi)
        cnt = jnp.sum((z >= mid).astype(jnp.float32), axis=1, keepdims=True)
        pred = cnt >= TOPK
        return jnp.where(pred, mid, lo), jnp.where(pred, hi, mid)

    lo, hi = jax.lax.fori_loop(0, BISECT_ITERS, body, (lo, hi))
    out_ref[...] = jnp.where(z >= lo, jnp.maximum(z, 0.0), 0.0)


def _dec_body(a_ref, b_ref, out_ref):
    k = pl.program_id(0)
    r = pl.program_id(1)
    rows = pl.ds(r * BM, BM)

    @pl.when(k == 0)
    def _():
        out_ref[rows, :] = jnp.zeros((BM, HIDDEN), jnp.float32)

    out_ref[rows, :] += jax.lax.dot_general(
        a_ref[...], b_ref[...],
        (((1,), (1,)), ((), ())),
        preferred_element_type=jnp.float32,
        precision=jax.lax.Precision.DEFAULT)


def kernel(x, W_enc, W_dec):
    z = pl.pallas_call(
        _enc_body,
        grid=(LATENT // LC, NTOK // BM),
        in_specs=[pl.BlockSpec((BM, HIDDEN), lambda j, r: (r, 0)),
                  pl.BlockSpec((LC, HIDDEN), lambda j, r: (j, 0))],
        out_specs=pl.BlockSpec((BM, LC), lambda j, r: (r, j)),
        out_shape=jax.ShapeDtypeStruct((NTOK, LATENT), jnp.float32),
    )(x, W_enc)
    z_sparse = pl.pallas_call(
        _topk_body,
        grid=(NTOK // BR,),
        in_specs=[pl.BlockSpec((BR, LATENT), lambda i: (i, 0))],
        out_specs=pl.BlockSpec((BR, LATENT), lambda i: (i, 0)),
        out_shape=jax.ShapeDtypeStruct((NTOK, LATENT), jnp.float32),
    )(z)
    x_hat = pl.pallas_call(
        _dec_body,
        grid=(LATENT // KC, NTOK // BM),
        in_specs=[pl.BlockSpec((BM, KC), lambda k, r: (r, k)),
                  pl.BlockSpec((HIDDEN, KC), lambda k, r: (0, k))],
        out_specs=pl.BlockSpec((NTOK, HIDDEN), lambda k, r: (0, 0)),
        out_shape=jax.ShapeDtypeStruct((NTOK, HIDDEN), jnp.float32),
    )(z_sparse, W_dec)
    return x_hat, z_sparse
